# Initial kernel scaffold; baseline (speedup 1.0000x reference)
import jax
import jax.numpy as jnp
from jax.experimental import pallas as pl


def kernel(x, w1p, b1p, w2p, b2p, w1c, b1c, w2c, b2c, w_lin, gamma, beta):
    raise NotImplementedError("write your pallas kernel here")


# speedup vs baseline: 1.2116x; 1.2116x over previous
import numpy as np

import jax
import jax.numpy as jnp
from jax.experimental import pallas as pl
from jax.experimental.pallas import tpu as pltpu

_EPS = 1e-3
_F32 = jnp.float32
_BF16 = jnp.bfloat16


def _consts(P, C):
    PC = P * C
    ip = np.arange(PC) // C
    ic = np.arange(PC) % C
    one_p = (ip[:, None] == np.arange(P)[None, :]).astype(np.float32)
    one_c = (ic[:, None] == np.arange(C)[None, :]).astype(np.float32)
    m_mean = np.concatenate([one_p / C, one_c / P], axis=1)
    s_pmax = one_p * (ic[:, None] == C - 1).astype(np.float32)
    b_p = one_p.T
    return (jnp.asarray(m_mean, _BF16), jnp.asarray(s_pmax, _BF16),
            jnp.asarray(b_p, _BF16))


def _gate_kernel(P, C, HP, HC):
    PC, NU, NH = P * C, P + C, HP + HC

    def body(x_ref, mmean_ref, spmax_ref, bp_ref, w1_ref, b1_ref, w2_ref, b2_ref,
             xg_ref):
        tn = x_ref.shape[0]
        xf = x_ref[...]
        xb = xf.astype(_BF16)

        means = jnp.dot(xb, mmean_ref[...], preferred_element_type=_F32)

        r = xf
        s = C // 2
        while s >= 1:
            r = jnp.maximum(r, jnp.concatenate([r[:, :s], r[:, :-s]], axis=1))
            s //= 2
        pmax = jnp.dot(r.astype(_BF16), spmax_ref[...],
                       preferred_element_type=_F32)

        m = xf
        w = PC // 2
        while w >= C:
            m = jnp.maximum(m[:, :w], m[:, w:2 * w])
            w //= 2
        cmax = m

        u = jnp.concatenate(
            [jnp.concatenate([pmax, cmax], axis=1), means], axis=0)
        h = jnp.maximum(jnp.dot(u, w1_ref[...],
                                preferred_element_type=_F32) + b1_ref[...], 0.0)
        a = jnp.dot(h, w2_ref[...], preferred_element_type=_F32) + b2_ref[...]
        scales = a[:tn] + a[tn:]

        sp_b = jnp.dot(scales[:, :P].astype(_BF16), bp_ref[...],
                       preferred_element_type=_F32)
        sc_b = jnp.concatenate([scales[:, P:]] * P, axis=1)
        g = jax.nn.sigmoid(sp_b * sc_b)
        xg_ref[...] = (xf * g).astype(_BF16)

    return body


def _linear_bn_kernel(P, C, OUT, PACK, tn):
    LANES = PACK * C
    ROWS_PER = P // PACK

    def body(xg_ref, w4_ref, mm_ref, st_ref):
        xg = xg_ref[...]
        y = jnp.dot(xg, w4_ref[...], preferred_element_type=_F32)

        ymax, ymin = y, y
        w = (PACK * OUT) // 2
        while w >= OUT:
            ymax = jnp.maximum(ymax[:, :w], ymax[:, w:2 * w])
            ymin = jnp.minimum(ymin[:, :w], ymin[:, w:2 * w])
            w //= 2
        vmax = jnp.max(ymax.reshape(tn, ROWS_PER, OUT), axis=1)
        vmin = jnp.min(ymin.reshape(tn, ROWS_PER, OUT), axis=1)
        mm_ref[...] = jnp.concatenate([vmax, vmin], axis=1)

        colsum = jnp.sum(y, axis=0, keepdims=True)
        tsum = colsum
        w = (PACK * OUT) // 2
        while w >= OUT:
            tsum = tsum[:, :w] + tsum[:, w:2 * w]
            w //= 2
        tmean = tsum * (1.0 / (tn * P))
        d = y - jnp.concatenate([tmean] * PACK, axis=1)
        m2col = jnp.sum(d * d, axis=0, keepdims=True)
        tm2 = m2col
        w = (PACK * OUT) // 2
        while w >= OUT:
            tm2 = tm2[:, :w] + tm2[:, w:2 * w]
            w //= 2
        st_ref[...] = jnp.concatenate([tsum, tm2], axis=1)[None]

    return body


def kernel(x, w1p, b1p, w2p, b2p, w1c, b1c, w2c, b2c, w_lin, gamma, beta):
    N, P, C = x.shape
    OUT = w_lin.shape[1]
    HP, HC = w1p.shape[1], w1c.shape[1]
    PC, NU, NH = P * C, P + C, HP + HC
    PACK = 256 // C if C <= 256 else 1
    LANES = PACK * C
    ROWS_PER = P // PACK

    tn = 256
    while N % tn:
        tn //= 2
    grid_n = N // tn

    m_mean, s_pmax, b_p = _consts(P, C)

    w1 = jnp.zeros((NU, NH), _F32).at[:P, :HP].set(w1p).at[P:, HP:].set(w1c)
    b1 = jnp.concatenate([b1p, b1c], axis=1)
    w2 = jnp.zeros((NH, NU), _F32).at[:HP, :P].set(w2p).at[HP:, P:].set(w2c)
    b2 = jnp.concatenate([b2p, b2c], axis=1)

    x_flat = x.reshape(N, PC).astype(_F32)

    xg = pl.pallas_call(
        _gate_kernel(P, C, HP, HC),
        out_shape=jax.ShapeDtypeStruct((N, PC), _BF16),
        grid=(grid_n,),
        in_specs=[
            pl.BlockSpec((tn, PC), lambda i: (i, 0)),
            pl.BlockSpec((PC, NU), lambda i: (0, 0)),
            pl.BlockSpec((PC, P), lambda i: (0, 0)),
            pl.BlockSpec((P, PC), lambda i: (0, 0)),
            pl.BlockSpec((NU, NH), lambda i: (0, 0)),
            pl.BlockSpec((1, NH), lambda i: (0, 0)),
            pl.BlockSpec((NH, NU), lambda i: (0, 0)),
            pl.BlockSpec((1, NU), lambda i: (0, 0)),
        ],
        out_specs=pl.BlockSpec((tn, PC), lambda i: (i, 0)),
        compiler_params=pltpu.CompilerParams(
            dimension_semantics=("parallel",),
            vmem_limit_bytes=64 * 1024 * 1024,
        ),
    )(x_flat, m_mean, s_pmax, b_p, w1, b1, w2, b2)

    xg_packed = xg.reshape(N * ROWS_PER, LANES)
    w4 = jnp.einsum("pq,co->pcqo", jnp.eye(PACK, dtype=_F32),
                    w_lin.astype(_F32)).reshape(LANES, PACK * OUT).astype(_BF16)

    mm, stats = pl.pallas_call(
        _linear_bn_kernel(P, C, OUT, PACK, tn),
        out_shape=(
            jax.ShapeDtypeStruct((N, 2 * OUT), _F32),
            jax.ShapeDtypeStruct((grid_n, 1, 2 * OUT), _F32),
        ),
        grid=(grid_n,),
        in_specs=[
            pl.BlockSpec((tn * ROWS_PER, LANES), lambda i: (i, 0)),
            pl.BlockSpec((LANES, PACK * OUT), lambda i: (0, 0)),
        ],
        out_specs=(
            pl.BlockSpec((tn, 2 * OUT), lambda i: (i, 0)),
            pl.BlockSpec((1, 1, 2 * OUT), lambda i: (i, 0, 0)),
        ),
        compiler_params=pltpu.CompilerParams(
            dimension_semantics=("parallel",),
            vmem_limit_bytes=64 * 1024 * 1024,
        ),
    )(xg_packed, w4)

    npts = tn * P
    tmean = stats[:, 0, :OUT] / npts
    tvar = stats[:, 0, OUT:] / npts
    mean = jnp.mean(tmean, axis=0)
    var = jnp.mean(tvar, axis=0) + jnp.mean(jnp.square(tmean - mean[None, :]), axis=0)
    scale = gamma.reshape(-1) * jax.lax.rsqrt(var + _EPS)
    shift = beta.reshape(-1) - mean * scale
    pre = jnp.where(scale >= 0.0, mm[:, :OUT], mm[:, OUT:]) * scale + shift
    return jnp.maximum(pre, 0.0).reshape(N, 1, OUT)
